# Initial kernel scaffold; baseline (speedup 1.0000x reference)
import functools

import jax
import jax.numpy as jnp
from jax import lax
from jax.experimental import pallas as pl
from jax.experimental.pallas import tpu as pltpu
from jax.experimental.pallas import tpu_sc as plsc

N_NODES = 10000
N_EDGES = 320000
D = 128

NC = 2
NS = 16
NW = NC * NS
CH = 128
NCHUNK = 80
EPT = NCHUNK * CH
E_PAD = NW * EPT
N_ACC = 10240
ROWS_PER_TILE_ZERO = N_ACC // NS
ROWS_PER_TILE_OUT = N_NODES // NS


def _mm_body(x_ref, w_ref, b_ref, o_ref):
    x = x_ref[...]
    w = w_ref[...]
    o_ref[...] = lax.dot_general(
        x, w, (((1,), (1,)), ((), ())), preferred_element_type=jnp.float32
    ) + b_ref[...][None, :]


def _linear(feat, W, b):
    blk = 2000
    grid = N_NODES // blk
    return pl.pallas_call(
        _mm_body,
        grid=(grid,),
        in_specs=[
            pl.BlockSpec((blk, D), lambda i: (i, 0)),
            pl.BlockSpec((D, D), lambda i: (0, 0)),
            pl.BlockSpec((D,), lambda i: (0,)),
        ],
        out_specs=pl.BlockSpec((blk, D), lambda i: (i, 0)),
        out_shape=jax.ShapeDtypeStruct((N_NODES, D), jnp.float32),
    )(feat, W, b)


def _combine_body(a_ref, b_ref, o_ref):
    o_ref[...] = a_ref[...] + b_ref[...]


def _combine(part):
    blk = 2000
    grid = N_NODES // blk
    off = N_NODES // blk
    return pl.pallas_call(
        _combine_body,
        grid=(grid,),
        in_specs=[
            pl.BlockSpec((blk, D), lambda i: (i, 0)),
            pl.BlockSpec((blk, D), lambda i: (i + off, 0)),
        ],
        out_specs=pl.BlockSpec((blk, D), lambda i: (i, 0)),
        out_shape=jax.ShapeDtypeStruct((N_NODES, D), jnp.float32),
    )(part, part)


def _sc_body(h_hbm, src_hbm, dst_hbm, part_hbm, src_v, dst_v, rows_v,
             acc_sh, sem):
    c = lax.axis_index("c")
    s = lax.axis_index("s")
    wid = c * NS + s

    def _zero_row(r, carry):
        for k in range(D // 16):
            rows_v[r, pl.ds(k * 16, 16)] = jnp.zeros((16,), jnp.float32)
        return carry
    lax.fori_loop(0, CH, _zero_row, 0)
    for k in range(ROWS_PER_TILE_ZERO // CH):
        pltpu.sync_copy(rows_v,
                        acc_sh.at[pl.ds(s * ROWS_PER_TILE_ZERO + k * CH, CH)])
    plsc.subcore_barrier()

    pltpu.sync_copy(src_hbm.at[wid], src_v)
    pltpu.sync_copy(dst_hbm.at[wid], dst_v)

    def _chunk(j, carry):
        pltpu.async_copy(h_hbm.at[src_v.at[j]], rows_v, sem).wait()
        pltpu.sync_copy(rows_v, acc_sh.at[dst_v.at[j]], add=True)
        return carry
    lax.fori_loop(0, NCHUNK, _chunk, 0)
    plsc.subcore_barrier()

    base = s * ROWS_PER_TILE_OUT
    pltpu.sync_copy(acc_sh.at[pl.ds(base, ROWS_PER_TILE_OUT)],
                    part_hbm.at[pl.ds(c * N_NODES + base, ROWS_PER_TILE_OUT)])


@functools.partial(
    pl.kernel,
    out_type=jax.ShapeDtypeStruct((NC * N_NODES, D), jnp.float32),
    mesh=plsc.VectorSubcoreMesh(
        core_axis_name="c", subcore_axis_name="s", num_cores=NC,
        num_subcores=NS),
    scratch_types=[
        pltpu.VMEM((NCHUNK, CH), jnp.int32),
        pltpu.VMEM((NCHUNK, CH), jnp.int32),
        pltpu.VMEM((CH, D), jnp.float32),
        pltpu.VMEM_SHARED((N_ACC, D), jnp.float32),
        pltpu.SemaphoreType.DMA,
    ],
)
def _sc_aggregate(h_hbm, src_hbm, dst_hbm, part_hbm, src_v, dst_v, rows_v,
                  acc_sh, sem):
    _sc_body(h_hbm, src_hbm, dst_hbm, part_hbm, src_v, dst_v, rows_v,
             acc_sh, sem)


def kernel(feat, edge_index, W, b):
    h = _linear(feat, W, b)

    pad_n = E_PAD - N_EDGES
    pad_src = (jnp.arange(pad_n, dtype=jnp.int32) * 37) % N_NODES
    pad_dst = N_NODES + (jnp.arange(pad_n, dtype=jnp.int32) % (N_ACC - N_NODES))
    src = jnp.concatenate([edge_index[0], pad_src]).reshape(NW, NCHUNK, CH)
    dst = jnp.concatenate([edge_index[1], pad_dst]).reshape(NW, NCHUNK, CH)

    part = _sc_aggregate(h, src, dst)
    return _combine(part)


# speedup vs baseline: 8.8023x; 8.8023x over previous
import functools

import jax
import jax.numpy as jnp
from jax import lax
from jax.experimental import pallas as pl
from jax.experimental.pallas import tpu as pltpu
from jax.experimental.pallas import tpu_sc as plsc

N_NODES = 10000
N_EDGES = 320000
D = 128

NC = 2
NS = 16
NW = NC * NS
CH = 128
NCHUNK = 80
EPT = NCHUNK * CH
E_PAD = NW * EPT
N_ACC = 10240
ROWS_PER_TILE_ZERO = N_ACC // NS
ROWS_PER_TILE_OUT = N_NODES // NS


def _mm_body(x_ref, w_ref, b_ref, o_ref):
    x = x_ref[...]
    w = w_ref[...]
    o_ref[...] = lax.dot_general(
        x, w, (((1,), (1,)), ((), ())), preferred_element_type=jnp.float32
    ) + b_ref[...][None, :]


def _linear(feat, W, b):
    blk = 2000
    grid = N_NODES // blk
    return pl.pallas_call(
        _mm_body,
        grid=(grid,),
        in_specs=[
            pl.BlockSpec((blk, D), lambda i: (i, 0)),
            pl.BlockSpec((D, D), lambda i: (0, 0)),
            pl.BlockSpec((D,), lambda i: (0,)),
        ],
        out_specs=pl.BlockSpec((blk, D), lambda i: (i, 0)),
        out_shape=jax.ShapeDtypeStruct((N_NODES, D), jnp.float32),
    )(feat, W, b)


def _combine_body(a_ref, b_ref, o_ref):
    o_ref[...] = a_ref[0] + b_ref[0]


def _combine(part):
    blk = 2000
    grid = N_NODES // blk
    p3 = part.reshape(NC, N_ACC, D)
    return pl.pallas_call(
        _combine_body,
        grid=(grid,),
        in_specs=[
            pl.BlockSpec((1, blk, D), lambda i: (0, i, 0)),
            pl.BlockSpec((1, blk, D), lambda i: (1, i, 0)),
        ],
        out_specs=pl.BlockSpec((blk, D), lambda i: (i, 0)),
        out_shape=jax.ShapeDtypeStruct((N_NODES, D), jnp.float32),
    )(p3, p3)


def _sc_body(h_hbm, src_hbm, dst_hbm, part_hbm, src_v, dst_v, rows_v,
             acc_sh, sem):
    c = lax.axis_index("c")
    s = lax.axis_index("s")
    wid = c * NS + s

    def _zero_row(r, carry):
        for k in range(D // 16):
            rows_v[r, pl.ds(k * 16, 16)] = jnp.zeros((16,), jnp.float32)
        return carry
    lax.fori_loop(0, CH, _zero_row, 0)
    for k in range(ROWS_PER_TILE_ZERO // CH):
        pltpu.sync_copy(rows_v,
                        acc_sh.at[pl.ds(s * ROWS_PER_TILE_ZERO + k * CH, CH)])
    plsc.subcore_barrier()

    pltpu.sync_copy(src_hbm.at[wid], src_v)
    pltpu.sync_copy(dst_hbm.at[wid], dst_v)

    def _chunk(j, carry):
        pltpu.async_copy(h_hbm.at[src_v.at[j]], rows_v, sem).wait()
        pltpu.sync_copy(rows_v, acc_sh.at[dst_v.at[j]], add=True)
        return carry
    lax.fori_loop(0, NCHUNK, _chunk, 0)
    plsc.subcore_barrier()

    base = s * ROWS_PER_TILE_ZERO
    pltpu.sync_copy(acc_sh.at[pl.ds(base, ROWS_PER_TILE_ZERO)],
                    part_hbm.at[pl.ds(c * N_ACC + base, ROWS_PER_TILE_ZERO)])


@functools.partial(
    pl.kernel,
    out_type=jax.ShapeDtypeStruct((NC * N_ACC, D), jnp.float32),
    mesh=plsc.VectorSubcoreMesh(
        core_axis_name="c", subcore_axis_name="s", num_cores=NC,
        num_subcores=NS),
    scratch_types=[
        pltpu.VMEM((NCHUNK, CH), jnp.int32),
        pltpu.VMEM((NCHUNK, CH), jnp.int32),
        pltpu.VMEM((CH, D), jnp.float32),
        pltpu.VMEM_SHARED((N_ACC, D), jnp.float32),
        pltpu.SemaphoreType.DMA,
    ],
)
def _sc_aggregate(h_hbm, src_hbm, dst_hbm, part_hbm, src_v, dst_v, rows_v,
                  acc_sh, sem):
    _sc_body(h_hbm, src_hbm, dst_hbm, part_hbm, src_v, dst_v, rows_v,
             acc_sh, sem)


def kernel(feat, edge_index, W, b):
    h = _linear(feat, W, b)

    pad_n = E_PAD - N_EDGES
    pad_src = (jnp.arange(pad_n, dtype=jnp.int32) * 37) % N_NODES
    pad_dst = N_NODES + (jnp.arange(pad_n, dtype=jnp.int32) % (N_ACC - N_NODES))
    src = jnp.concatenate([edge_index[0], pad_src]).reshape(NW, NCHUNK, CH)
    dst = jnp.concatenate([edge_index[1], pad_dst]).reshape(NW, NCHUNK, CH)

    part = _sc_aggregate(h, src, dst)
    return _combine(part)


# speedup vs baseline: 12.6282x; 1.4347x over previous
import functools

import jax
import jax.numpy as jnp
from jax import lax
from jax.experimental import pallas as pl
from jax.experimental.pallas import tpu as pltpu
from jax.experimental.pallas import tpu_sc as plsc

N_NODES = 10000
N_EDGES = 320000
D = 128

NC = 2
NS = 16
NW = NC * NS
CH = 128
NCHUNK = 80
HC = 40
EPT = NCHUNK * CH
E_PAD = NW * EPT
N_ACC = 10240
ROWS_PER_TILE_ZERO = N_ACC // NS
ROWS_PER_TILE_OUT = N_NODES // NS


def _mm_body(x_ref, w_ref, b_ref, o_ref):
    x = x_ref[...]
    w = w_ref[...]
    o_ref[...] = lax.dot_general(
        x, w, (((1,), (1,)), ((), ())), preferred_element_type=jnp.float32
    ) + b_ref[...][None, :]


def _linear(feat, W, b):
    blk = 2000
    grid = N_NODES // blk
    return pl.pallas_call(
        _mm_body,
        grid=(grid,),
        in_specs=[
            pl.BlockSpec((blk, D), lambda i: (i, 0)),
            pl.BlockSpec((D, D), lambda i: (0, 0)),
            pl.BlockSpec((D,), lambda i: (0,)),
        ],
        out_specs=pl.BlockSpec((blk, D), lambda i: (i, 0)),
        out_shape=jax.ShapeDtypeStruct((N_NODES, D), jnp.float32),
    )(feat, W, b)


def _combine_body(a_ref, b_ref, o_ref):
    o_ref[...] = a_ref[0] + b_ref[0]


def _combine(part):
    blk = 2000
    grid = N_NODES // blk
    p3 = part.reshape(NC, N_ACC, D)
    return pl.pallas_call(
        _combine_body,
        grid=(grid,),
        in_specs=[
            pl.BlockSpec((1, blk, D), lambda i: (0, i, 0)),
            pl.BlockSpec((1, blk, D), lambda i: (1, i, 0)),
        ],
        out_specs=pl.BlockSpec((blk, D), lambda i: (i, 0)),
        out_shape=jax.ShapeDtypeStruct((N_NODES, D), jnp.float32),
    )(p3, p3)


def _sc_body(h_hbm, src_hbm, dst_hbm, part_hbm, src_v, dst_v, rows_v,
             rows2_v, acc_sh, sem, sem2):
    c = lax.axis_index("c")
    s = lax.axis_index("s")
    wid = c * NS + s

    def _zero_row(r, carry):
        for k in range(D // 16):
            rows_v[r, pl.ds(k * 16, 16)] = jnp.zeros((16,), jnp.float32)
        return carry
    lax.fori_loop(0, CH, _zero_row, 0)
    for k in range(ROWS_PER_TILE_ZERO // CH):
        pltpu.sync_copy(rows_v,
                        acc_sh.at[pl.ds(s * ROWS_PER_TILE_ZERO + k * CH, CH)])
    plsc.subcore_barrier()

    for p in range(NCHUNK // HC):
        pltpu.sync_copy(src_hbm.at[wid, pl.ds(p * HC, HC)], src_v)
        pltpu.sync_copy(dst_hbm.at[wid, pl.ds(p * HC, HC)], dst_v)
        pltpu.async_copy(h_hbm.at[src_v.at[0]], rows_v, sem)

        def _pair(j, carry):
            a = 2 * j
            pltpu.async_copy(h_hbm.at[src_v.at[a + 1]], rows2_v, sem2)
            pltpu.make_async_copy(h_hbm.at[src_v.at[a]], rows_v, sem).wait()
            pltpu.sync_copy(rows_v, acc_sh.at[dst_v.at[a]], add=True)

            @pl.when(j + 1 < HC // 2)
            def _():
                pltpu.async_copy(h_hbm.at[src_v.at[a + 2]], rows_v, sem)
            pltpu.make_async_copy(h_hbm.at[src_v.at[a + 1]], rows2_v,
                                  sem2).wait()
            pltpu.sync_copy(rows2_v, acc_sh.at[dst_v.at[a + 1]], add=True)
            return carry
        lax.fori_loop(0, HC // 2, _pair, 0)
    plsc.subcore_barrier()

    base = s * ROWS_PER_TILE_ZERO
    pltpu.sync_copy(acc_sh.at[pl.ds(base, ROWS_PER_TILE_ZERO)],
                    part_hbm.at[pl.ds(c * N_ACC + base, ROWS_PER_TILE_ZERO)])


@functools.partial(
    pl.kernel,
    out_type=jax.ShapeDtypeStruct((NC * N_ACC, D), jnp.float32),
    mesh=plsc.VectorSubcoreMesh(
        core_axis_name="c", subcore_axis_name="s", num_cores=NC,
        num_subcores=NS),
    scratch_types=[
        pltpu.VMEM((HC, CH), jnp.int32),
        pltpu.VMEM((HC, CH), jnp.int32),
        pltpu.VMEM((CH, D), jnp.float32),
        pltpu.VMEM((CH, D), jnp.float32),
        pltpu.VMEM_SHARED((N_ACC, D), jnp.float32),
        pltpu.SemaphoreType.DMA,
        pltpu.SemaphoreType.DMA,
    ],
)
def _sc_aggregate(h_hbm, src_hbm, dst_hbm, part_hbm, src_v, dst_v, rows_v,
                  rows2_v, acc_sh, sem, sem2):
    _sc_body(h_hbm, src_hbm, dst_hbm, part_hbm, src_v, dst_v, rows_v,
             rows2_v, acc_sh, sem, sem2)


def kernel(feat, edge_index, W, b):
    h = _linear(feat, W, b)

    pad_n = E_PAD - N_EDGES
    pad_src = (jnp.arange(pad_n, dtype=jnp.int32) * 37) % N_NODES
    pad_dst = N_NODES + (jnp.arange(pad_n, dtype=jnp.int32) % (N_ACC - N_NODES))
    src = jnp.concatenate([edge_index[0], pad_src]).reshape(NW, NCHUNK, CH)
    dst = jnp.concatenate([edge_index[1], pad_dst]).reshape(NW, NCHUNK, CH)

    part = _sc_aggregate(h, src, dst)
    return _combine(part)
